# Initial kernel scaffold; baseline (speedup 1.0000x reference)
import jax
import jax.numpy as jnp
from jax.experimental import pallas as pl


def kernel(x, edge_index, W_mu, b_mu, W_logstd, b_logstd):
    raise NotImplementedError("write your pallas kernel here")


# speedup vs baseline: 15.8556x; 15.8556x over previous
import functools

import jax
import jax.numpy as jnp
from jax import lax
from jax.experimental import pallas as pl
from jax.experimental.pallas import tpu as pltpu
from jax.experimental.pallas import tpu_sc as plsc

N = 10000
E = 320000
D = 128
NH = N // 2
CR = 5120
CRT = CR // 16
ASLAB = 80
NC = 2
NS = 16
CH = 128
DEGR = 10240
DRT = DEGR // 16
DSLAB = 80

EPT_D = E // (NC * NS)
NCH_D = -(-EPT_D // CH)
PAD_D = NCH_D * CH
TAIL_D = PAD_D - EPT_D

EPT_A = E // NS
NCH_A = -(-EPT_A // CH)
PAD_A = NCH_A * CH
TAIL_A = PAD_A - EPT_A


def _deg_body(dst_hbm, out_hbm, flat_v, idx2_v, ones_v, buf_v, deg_sh):
    c = lax.axis_index("c")
    s = lax.axis_index("s")
    ebase = c * (E // NC) + s * EPT_D
    zvec = jnp.zeros((16,), jnp.float32)

    def zrow(i, _):
        def zk(k, __):
            buf_v[i, pl.ds(k * 16, 16)] = zvec
            return 0

        lax.fori_loop(0, D // 16, zk, 0)
        return 0

    lax.fori_loop(0, DSLAB, zrow, 0)

    def zslab(q, _):
        pltpu.sync_copy(buf_v, deg_sh.at[pl.ds(s * DRT + q * DSLAB, DSLAB)])
        return 0

    lax.fori_loop(0, DRT // DSLAB, zslab, 0)
    plsc.subcore_barrier()

    ovec = jnp.ones((16,), jnp.float32)

    def orow(i, _):
        def ok(k, __):
            ones_v[i, pl.ds(k * 16, 16)] = ovec
            return 0

        lax.fori_loop(0, D // 16, ok, 0)
        return 0

    lax.fori_loop(0, CH, orow, 0)

    nvec = jnp.full((16,), N, jnp.int32)

    def trow(i, _):
        flat_v[pl.ds(EPT_D + i * 16, 16)] = nvec
        return 0

    lax.fori_loop(0, TAIL_D // 16, trow, 0)
    pltpu.sync_copy(dst_hbm.at[pl.ds(ebase, EPT_D)], flat_v.at[pl.ds(0, EPT_D)])

    def reshape_row(j, _):
        def mv(k, __):
            idx2_v[j, pl.ds(k * 16, 16)] = flat_v[pl.ds(j * CH + k * 16, 16)]
            return 0

        lax.fori_loop(0, CH // 16, mv, 0)
        return 0

    lax.fori_loop(0, NCH_D, reshape_row, 0)

    def scat(j, _):
        pltpu.sync_copy(ones_v, deg_sh.at[idx2_v.at[j]], add=True)
        return 0

    lax.fori_loop(0, NCH_D, scat, 0)
    plsc.subcore_barrier()

    def oslab(q, _):
        pltpu.sync_copy(deg_sh.at[pl.ds(s * DRT + q * DSLAB, DSLAB)], buf_v)
        pltpu.sync_copy(buf_v, out_hbm.at[c, pl.ds(s * DRT + q * DSLAB, DSLAB)])
        return 0

    lax.fori_loop(0, DRT // DSLAB, oslab, 0)


_deg_call = functools.partial(
    pl.kernel,
    out_type=jax.ShapeDtypeStruct((NC, DEGR, D), jnp.float32),
    mesh=plsc.VectorSubcoreMesh(core_axis_name="c", subcore_axis_name="s"),
    scratch_types=[
        pltpu.VMEM((PAD_D,), jnp.int32),
        pltpu.VMEM((NCH_D, CH), jnp.int32),
        pltpu.VMEM((CH, D), jnp.float32),
        pltpu.VMEM((DSLAB, D), jnp.float32),
        pltpu.VMEM_SHARED((DEGR, D), jnp.float32),
    ],
)(_deg_body)


def _agg_body(src_hbm, dst_hbm, xs_hbm, out_hbm,
              fsrc_v, fdst_v, idst2_v, rows_v, buf_v, s_sh):
    c = lax.axis_index("c")
    s = lax.axis_index("s")
    ebase = s * EPT_A
    zvec = jnp.zeros((16,), jnp.float32)

    def zrow(i, _):
        def zk(k, __):
            buf_v[i, pl.ds(k * 16, 16)] = zvec
            return 0

        lax.fori_loop(0, D // 16, zk, 0)
        return 0

    lax.fori_loop(0, ASLAB, zrow, 0)

    def zslab(q, _):
        pltpu.sync_copy(buf_v, s_sh.at[pl.ds(s * CRT + q * ASLAB, ASLAB)])
        return 0

    lax.fori_loop(0, CRT // ASLAB, zslab, 0)
    plsc.subcore_barrier()

    zidx = jnp.zeros((16,), jnp.int32)
    dvec = jnp.full((16,), NH, jnp.int32)

    def trow(i, _):
        fsrc_v[pl.ds(EPT_A + i * 16, 16)] = zidx
        fdst_v[pl.ds(EPT_A + i * 16, 16)] = dvec + (c * NH)
        return 0

    lax.fori_loop(0, TAIL_A // 16, trow, 0)
    pltpu.sync_copy(src_hbm.at[pl.ds(ebase, EPT_A)], fsrc_v.at[pl.ds(0, EPT_A)])
    pltpu.sync_copy(dst_hbm.at[pl.ds(ebase, EPT_A)], fdst_v.at[pl.ds(0, EPT_A)])

    base = jnp.full((16,), 1, jnp.int32) * (c * NH)
    dump = jnp.full((16,), NH, jnp.int32)

    def reshape_row(j, _):
        def mv(k, __):
            dv = fdst_v[pl.ds(j * CH + k * 16, 16)]
            lo = dv - base
            ok = (lo >= 0) & (lo < NH)
            idst2_v[j, pl.ds(k * 16, 16)] = jnp.where(ok, lo, dump)
            return 0

        lax.fori_loop(0, CH // 16, mv, 0)
        return 0

    lax.fori_loop(0, NCH_A, reshape_row, 0)

    def chunk(j, _):
        pltpu.sync_copy(xs_hbm.at[fsrc_v.at[pl.ds(j * CH, CH)]], rows_v)
        pltpu.sync_copy(rows_v, s_sh.at[idst2_v.at[j]], add=True)
        return 0

    lax.fori_loop(0, NCH_A, chunk, 0)
    plsc.subcore_barrier()

    def oslab(q, _):
        pltpu.sync_copy(s_sh.at[pl.ds(s * CRT + q * ASLAB, ASLAB)], buf_v)
        pltpu.sync_copy(buf_v, out_hbm.at[c, pl.ds(s * CRT + q * ASLAB, ASLAB)])
        return 0

    lax.fori_loop(0, CRT // ASLAB, oslab, 0)


_agg_call = functools.partial(
    pl.kernel,
    out_type=jax.ShapeDtypeStruct((NC, CR, D), jnp.float32),
    mesh=plsc.VectorSubcoreMesh(core_axis_name="c", subcore_axis_name="s"),
    scratch_types=[
        pltpu.VMEM((PAD_A,), jnp.int32),
        pltpu.VMEM((PAD_A,), jnp.int32),
        pltpu.VMEM((NCH_A, CH), jnp.int32),
        pltpu.VMEM((CH, D), jnp.float32),
        pltpu.VMEM((ASLAB, D), jnp.float32),
        pltpu.VMEM_SHARED((CR, D), jnp.float32),
    ],
)(_agg_body)

_RB = 1000


def _scale_body(deg_ref, x_ref, xs_ref):
    d = deg_ref[:, 0:1] + deg_ref[:, 1:2] + 1.0
    xs_ref[:, :] = x_ref[:, :] * lax.rsqrt(d)


_scale_call = pl.pallas_call(
    _scale_body,
    grid=(N // _RB,),
    in_specs=[
        pl.BlockSpec((_RB, NC), lambda i: (i, 0)),
        pl.BlockSpec((_RB, D), lambda i: (i, 0)),
    ],
    out_specs=pl.BlockSpec((_RB, D), lambda i: (i, 0)),
    out_shape=jax.ShapeDtypeStruct((N, D), jnp.float32),
)


def _mm_body(s_ref, xs_ref, deg_ref, wmu_ref, bmu_ref, wls_ref, bls_ref,
             mu_ref, ls_ref):
    d = deg_ref[:, 0:1] + deg_ref[:, 1:2] + 1.0
    agg = (s_ref[0] + xs_ref[:, :]) * lax.rsqrt(d)
    mu_ref[:, :] = (
        jnp.dot(agg, wmu_ref[:, :], preferred_element_type=jnp.float32)
        + bmu_ref[:, :]
    )
    ls_ref[:, :] = (
        jnp.dot(agg, wls_ref[:, :], preferred_element_type=jnp.float32)
        + bls_ref[:, :]
    )


_mm_call = pl.pallas_call(
    _mm_body,
    grid=(N // _RB,),
    in_specs=[
        pl.BlockSpec((1, _RB, D), lambda i: (i // 5, i % 5, 0)),
        pl.BlockSpec((_RB, D), lambda i: (i, 0)),
        pl.BlockSpec((_RB, NC), lambda i: (i, 0)),
        pl.BlockSpec((D, D), lambda i: (0, 0)),
        pl.BlockSpec((1, D), lambda i: (0, 0)),
        pl.BlockSpec((D, D), lambda i: (0, 0)),
        pl.BlockSpec((1, D), lambda i: (0, 0)),
    ],
    out_specs=[
        pl.BlockSpec((_RB, D), lambda i: (i, 0)),
        pl.BlockSpec((_RB, D), lambda i: (i, 0)),
    ],
    out_shape=[
        jax.ShapeDtypeStruct((N, D), jnp.float32),
        jax.ShapeDtypeStruct((N, D), jnp.float32),
    ],
)


def kernel(x, edge_index, W_mu, b_mu, W_logstd, b_logstd):
    ei = edge_index.astype(jnp.int32)
    src = ei[0]
    dst = ei[1]
    deg2 = _deg_call(dst)
    dega = deg2[:, :N, 0].transpose(1, 0)
    xs = _scale_call(dega, x)
    s2 = _agg_call(src, dst, xs)
    mu, logstd = _mm_call(s2, xs, dega, W_mu, b_mu.reshape(1, D),
                          W_logstd, b_logstd.reshape(1, D))
    return (mu, logstd)


# speedup vs baseline: 16.0298x; 1.0110x over previous
import functools

import jax
import jax.numpy as jnp
from jax import lax
from jax.experimental import pallas as pl
from jax.experimental.pallas import tpu as pltpu
from jax.experimental.pallas import tpu_sc as plsc

N = 10000
E = 320000
D = 128
NH = N // 2
CR = 5120
CRT = CR // 16
ASLAB = 80
NC = 2
NS = 16
CH = 128
DEGR = 10240
DRT = DEGR // 16
DSLAB = 80

EPT_D = E // (NC * NS)
NCH_D = -(-EPT_D // CH)
PAD_D = NCH_D * CH
TAIL_D = PAD_D - EPT_D

EPT_A = E // NS
NCH_A = -(-EPT_A // CH)
PAD_A = NCH_A * CH
TAIL_A = PAD_A - EPT_A


def _deg_body(dst_hbm, out_hbm, flat_v, idx2_v, ones_v, buf_v, deg_sh):
    c = lax.axis_index("c")
    s = lax.axis_index("s")
    ebase = c * (E // NC) + s * EPT_D
    zvec = jnp.zeros((16,), jnp.float32)

    def zrow(i, _):
        def zk(k, __):
            buf_v[i, pl.ds(k * 16, 16)] = zvec
            return 0

        lax.fori_loop(0, D // 16, zk, 0)
        return 0

    lax.fori_loop(0, DSLAB, zrow, 0)

    def zslab(q, _):
        pltpu.sync_copy(buf_v, deg_sh.at[pl.ds(s * DRT + q * DSLAB, DSLAB)])
        return 0

    lax.fori_loop(0, DRT // DSLAB, zslab, 0)
    plsc.subcore_barrier()

    ovec = jnp.ones((16,), jnp.float32)

    def orow(i, _):
        def ok(k, __):
            ones_v[i, pl.ds(k * 16, 16)] = ovec
            return 0

        lax.fori_loop(0, D // 16, ok, 0)
        return 0

    lax.fori_loop(0, CH, orow, 0)

    nvec = jnp.full((16,), N, jnp.int32)

    def trow(i, _):
        flat_v[pl.ds(EPT_D + i * 16, 16)] = nvec
        return 0

    lax.fori_loop(0, TAIL_D // 16, trow, 0)
    pltpu.sync_copy(dst_hbm.at[pl.ds(ebase, EPT_D)], flat_v.at[pl.ds(0, EPT_D)])

    def reshape_row(j, _):
        def mv(k, __):
            idx2_v[j, pl.ds(k * 16, 16)] = flat_v[pl.ds(j * CH + k * 16, 16)]
            return 0

        lax.fori_loop(0, CH // 16, mv, 0)
        return 0

    lax.fori_loop(0, NCH_D, reshape_row, 0)

    def scat(j, _):
        pltpu.sync_copy(ones_v, deg_sh.at[idx2_v.at[j]], add=True)
        return 0

    lax.fori_loop(0, NCH_D, scat, 0)
    plsc.subcore_barrier()

    def oslab(q, _):
        pltpu.sync_copy(deg_sh.at[pl.ds(s * DRT + q * DSLAB, DSLAB)], buf_v)
        pltpu.sync_copy(buf_v, out_hbm.at[c, pl.ds(s * DRT + q * DSLAB, DSLAB)])
        return 0

    lax.fori_loop(0, DRT // DSLAB, oslab, 0)


_deg_call = functools.partial(
    pl.kernel,
    out_type=jax.ShapeDtypeStruct((NC, DEGR, D), jnp.float32),
    mesh=plsc.VectorSubcoreMesh(core_axis_name="c", subcore_axis_name="s"),
    scratch_types=[
        pltpu.VMEM((PAD_D,), jnp.int32),
        pltpu.VMEM((NCH_D, CH), jnp.int32),
        pltpu.VMEM((CH, D), jnp.float32),
        pltpu.VMEM((DSLAB, D), jnp.float32),
        pltpu.VMEM_SHARED((DEGR, D), jnp.float32),
    ],
)(_deg_body)


def _agg_body(src_hbm, dst_hbm, xs_hbm, out_hbm,
              fsrc_v, fdst_v, idst2_v, rows_v, buf_v, s_sh):
    c = lax.axis_index("c")
    s = lax.axis_index("s")
    ebase = s * EPT_A
    zvec = jnp.zeros((16,), jnp.float32)

    def zrow(i, _):
        def zk(k, __):
            buf_v[i, pl.ds(k * 16, 16)] = zvec
            return 0

        lax.fori_loop(0, D // 16, zk, 0)
        return 0

    lax.fori_loop(0, ASLAB, zrow, 0)

    def zslab(q, _):
        pltpu.sync_copy(buf_v, s_sh.at[pl.ds(s * CRT + q * ASLAB, ASLAB)])
        return 0

    lax.fori_loop(0, CRT // ASLAB, zslab, 0)
    plsc.subcore_barrier()

    zidx = jnp.zeros((16,), jnp.int32)
    dvec = jnp.full((16,), NH, jnp.int32)

    def trow(i, _):
        fsrc_v[pl.ds(EPT_A + i * 16, 16)] = zidx
        fdst_v[pl.ds(EPT_A + i * 16, 16)] = dvec + (c * NH)
        return 0

    lax.fori_loop(0, TAIL_A // 16, trow, 0)
    pltpu.sync_copy(src_hbm.at[pl.ds(ebase, EPT_A)], fsrc_v.at[pl.ds(0, EPT_A)])
    pltpu.sync_copy(dst_hbm.at[pl.ds(ebase, EPT_A)], fdst_v.at[pl.ds(0, EPT_A)])

    base = jnp.full((16,), 1, jnp.int32) * (c * NH)
    dump = jnp.full((16,), NH, jnp.int32)
    m63 = jnp.full((16,), 63, jnp.int32)

    def reshape_row(j, _):
        def mv(k, __):
            dv = fdst_v[pl.ds(j * CH + k * 16, 16)]
            lo = dv - base
            ok = (lo >= 0) & (lo < NH)
            idst2_v[j, pl.ds(k * 16, 16)] = jnp.where(ok, lo, dump + (dv & m63))
            return 0

        lax.fori_loop(0, CH // 16, mv, 0)
        return 0

    lax.fori_loop(0, NCH_A, reshape_row, 0)

    def chunk(j, _):
        pltpu.sync_copy(xs_hbm.at[fsrc_v.at[pl.ds(j * CH, CH)]], rows_v)
        pltpu.sync_copy(rows_v, s_sh.at[idst2_v.at[j]], add=True)
        return 0

    lax.fori_loop(0, NCH_A, chunk, 0)
    plsc.subcore_barrier()

    def oslab(q, _):
        pltpu.sync_copy(s_sh.at[pl.ds(s * CRT + q * ASLAB, ASLAB)], buf_v)
        pltpu.sync_copy(buf_v, out_hbm.at[c, pl.ds(s * CRT + q * ASLAB, ASLAB)])
        return 0

    lax.fori_loop(0, CRT // ASLAB, oslab, 0)


_agg_call = functools.partial(
    pl.kernel,
    out_type=jax.ShapeDtypeStruct((NC, CR, D), jnp.float32),
    mesh=plsc.VectorSubcoreMesh(core_axis_name="c", subcore_axis_name="s"),
    scratch_types=[
        pltpu.VMEM((PAD_A,), jnp.int32),
        pltpu.VMEM((PAD_A,), jnp.int32),
        pltpu.VMEM((NCH_A, CH), jnp.int32),
        pltpu.VMEM((CH, D), jnp.float32),
        pltpu.VMEM((ASLAB, D), jnp.float32),
        pltpu.VMEM_SHARED((CR, D), jnp.float32),
    ],
)(_agg_body)

_RB = 1000


def _scale_body(deg_ref, x_ref, xs_ref):
    d = deg_ref[:, 0:1] + deg_ref[:, 1:2] + 1.0
    xs_ref[:, :] = x_ref[:, :] * lax.rsqrt(d)


_scale_call = pl.pallas_call(
    _scale_body,
    grid=(N // _RB,),
    in_specs=[
        pl.BlockSpec((_RB, NC), lambda i: (i, 0)),
        pl.BlockSpec((_RB, D), lambda i: (i, 0)),
    ],
    out_specs=pl.BlockSpec((_RB, D), lambda i: (i, 0)),
    out_shape=jax.ShapeDtypeStruct((N, D), jnp.float32),
)


def _mm_body(s_ref, xs_ref, deg_ref, wmu_ref, bmu_ref, wls_ref, bls_ref,
             mu_ref, ls_ref):
    d = deg_ref[:, 0:1] + deg_ref[:, 1:2] + 1.0
    agg = (s_ref[0] + xs_ref[:, :]) * lax.rsqrt(d)
    mu_ref[:, :] = (
        jnp.dot(agg, wmu_ref[:, :], preferred_element_type=jnp.float32)
        + bmu_ref[:, :]
    )
    ls_ref[:, :] = (
        jnp.dot(agg, wls_ref[:, :], preferred_element_type=jnp.float32)
        + bls_ref[:, :]
    )


_mm_call = pl.pallas_call(
    _mm_body,
    grid=(N // _RB,),
    in_specs=[
        pl.BlockSpec((1, _RB, D), lambda i: (i // 5, i % 5, 0)),
        pl.BlockSpec((_RB, D), lambda i: (i, 0)),
        pl.BlockSpec((_RB, NC), lambda i: (i, 0)),
        pl.BlockSpec((D, D), lambda i: (0, 0)),
        pl.BlockSpec((1, D), lambda i: (0, 0)),
        pl.BlockSpec((D, D), lambda i: (0, 0)),
        pl.BlockSpec((1, D), lambda i: (0, 0)),
    ],
    out_specs=[
        pl.BlockSpec((_RB, D), lambda i: (i, 0)),
        pl.BlockSpec((_RB, D), lambda i: (i, 0)),
    ],
    out_shape=[
        jax.ShapeDtypeStruct((N, D), jnp.float32),
        jax.ShapeDtypeStruct((N, D), jnp.float32),
    ],
)


def kernel(x, edge_index, W_mu, b_mu, W_logstd, b_logstd):
    ei = edge_index.astype(jnp.int32)
    src = ei[0]
    dst = ei[1]
    deg2 = _deg_call(dst)
    dega = deg2[:, :N, 0].transpose(1, 0)
    xs = _scale_call(dega, x)
    s2 = _agg_call(src, dst, xs)
    mu, logstd = _mm_call(s2, xs, dega, W_mu, b_mu.reshape(1, D),
                          W_logstd, b_logstd.reshape(1, D))
    return (mu, logstd)
